# Initial kernel scaffold; baseline (speedup 1.0000x reference)
import jax
import jax.numpy as jnp
from jax.experimental import pallas as pl


def kernel(data, warp):
    raise NotImplementedError("write your pallas kernel here")


# speedup vs baseline: 56.0517x; 56.0517x over previous
import functools

import jax
import jax.numpy as jnp
from jax import lax
from jax.experimental import pallas as pl
from jax.experimental.pallas import tpu as pltpu
from jax.experimental.pallas import tpu_sc as plsc

B = 2
N = 64
C = 8
NQ = B * N * N * N
NROWS = NQ
LOGV = 18

NC = 2
NS = 16
NW = NC * NS
QPW = NQ // NW
CHUNK = 128
NCHUNK = QPW // CHUNK
GPC = CHUNK // 16
IDXN = CHUNK * 8
NDMA = 8


def _sc_body(table, warpf, out, wch, idxb, wgt, rows, outb, sem0, sem1):
    wid = lax.axis_index("s") * NC + lax.axis_index("c")
    qw0 = wid * QPW
    iota = lax.iota(jnp.int32, 16)
    fiota = iota.astype(jnp.float32) * 0.0
    sems = (sem0, sem1)

    def load_warp(buf, g):
        base = (qw0 + g * CHUNK) * 3
        pltpu.sync_copy(warpf.at[pl.ds(base, CHUNK * 3)], wch.at[buf])

    def phase_a(buf, g):
        qchunk0 = qw0 + g * CHUNK
        bsplat = iota * 0 + buf

        def grp(g16, _):
            qoff = g16 * 16
            qv = iota + qoff
            q3 = qv * 3
            w0 = plsc.load_gather(wch, [bsplat, q3])
            w1 = plsc.load_gather(wch, [bsplat, q3 + 1])
            w2 = plsc.load_gather(wch, [bsplat, q3 + 2])
            c0 = w0.astype(jnp.int32)
            c1 = w1.astype(jnp.int32)
            c2 = w2.astype(jnp.int32)
            d0 = w0 - c0.astype(jnp.float32)
            d1 = w1 - c1.astype(jnp.float32)
            d2 = w2 - c2.astype(jnp.float32)
            e0 = 1.0 - d0
            e1 = 1.0 - d1
            e2 = 1.0 - d2
            qg = qv + qchunk0
            bb = lax.shift_left(lax.shift_right_logical(qg, LOGV), LOGV)
            r = bb + lax.shift_left(c0, 12) + lax.shift_left(c1, 6) + c2
            pos0 = qv * 8
            vals = (r, r + 1, r + 64, r + 65, r + 4096, r + 4097,
                    r + 4160, r + 4161)
            ws = (e0 * e1 * e2, e0 * e1 * d2, e0 * d1 * e2, e0 * d1 * d2,
                  d0 * e1 * e2, d0 * e1 * d2, d0 * d1 * e2, d0 * d1 * d2)
            wbase = buf * 8 * CHUNK + qoff
            for c in range(8):
                p = pos0 + c
                plsc.store_scatter(
                    idxb, [bsplat, lax.shift_right_logical(p, 7),
                           lax.bitwise_and(p, 127)], vals[c])
                wgt[pl.ds(wbase + c * CHUNK, 16)] = ws[c]
            return 0

        lax.fori_loop(0, GPC, grp, 0, unroll=False)

    def fire_gathers(buf):
        ib = idxb.at[buf]
        rb = rows.at[buf]
        for j in range(NDMA):
            pltpu.make_async_copy(
                table.at[ib.at[j]],
                rb.at[pl.ds(j * 128, 128)],
                sems[buf]).start()

    def drain_gathers(buf):
        pltpu.make_async_copy(
            table.at[pl.ds(0, IDXN)], rows.at[buf], sems[buf]).wait()

    def phase_b(buf, g):
        bsplat = iota * 0 + buf

        def grp(g16, _):
            qoff = g16 * 16
            qv = iota + qoff
            rowv0 = qv * 8
            wbase = buf * 8 * CHUNK + qoff
            wvs = [wgt[pl.ds(wbase + c * CHUNK, 16)] for c in range(8)]
            for ch in range(C):
                colv = iota * 0 + ch
                acc = fiota
                for c in range(8):
                    v = plsc.load_gather(rows, [bsplat, rowv0 + c, colv])
                    acc = acc + v * wvs[c]
                plsc.store_scatter(outb, [bsplat, rowv0 + ch], acc)
            return 0

        lax.fori_loop(0, GPC, grp, 0, unroll=False)
        base = (qw0 + g * CHUNK) * C
        pltpu.sync_copy(outb.at[buf], out.at[pl.ds(base, CHUNK * C)])

    load_warp(0, 0)
    phase_a(0, 0)
    fire_gathers(0)

    def step(g, _):
        for b in range(2):
            gg = g * 2 + b

            @pl.when(gg + 1 < NCHUNK)
            def _():
                load_warp(1 - b, gg + 1)
                phase_a(1 - b, gg + 1)
                fire_gathers(1 - b)

            drain_gathers(b)
            phase_b(b, gg)
        return 0

    lax.fori_loop(0, NCHUNK // 2, step, 0, unroll=False)


@jax.jit
def kernel(data, warp):
    table = data.reshape(NROWS, C)
    warpf = warp.reshape(NQ * 3)
    mesh = plsc.VectorSubcoreMesh(core_axis_name="c", subcore_axis_name="s")
    out = pl.kernel(
        _sc_body,
        out_type=jax.ShapeDtypeStruct((NQ * C,), jnp.float32),
        mesh=mesh,
        compiler_params=pltpu.CompilerParams(
            use_tc_tiling_on_sc=False, needs_layout_passes=False),
        scratch_types=[
            pltpu.VMEM((2, CHUNK * 3), jnp.float32),
            pltpu.VMEM((2, NDMA, 128), jnp.int32),
            pltpu.VMEM((2 * 8 * CHUNK,), jnp.float32),
            pltpu.VMEM((2, IDXN, C), jnp.float32),
            pltpu.VMEM((2, CHUNK * C), jnp.float32),
            pltpu.SemaphoreType.DMA,
            pltpu.SemaphoreType.DMA,
        ],
    )(table, warpf)
    return out.reshape(B, N, N, N, C)
